# baseline (speedup 1.0000x reference)
import functools

import jax
import jax.numpy as jnp
from jax import lax
from jax.experimental import pallas as pl
from jax.experimental.pallas import tpu as pltpu
from jax.experimental.pallas import tpu_sc as plsc

DIM = 64
BATCH = 16384

_NC = 2
_NS = 16
_L = 16
_NW = _NC * _NS
_B_PER_W = BATCH // _NW
_CHUNK = 128
_NCHUNK = _B_PER_W // _CHUNK

_mesh = plsc.VectorSubcoreMesh(core_axis_name="c", subcore_axis_name="s")


@functools.partial(
    pl.kernel,
    mesh=_mesh,
    out_type=jax.ShapeDtypeStruct((BATCH, DIM), jnp.float32),
    scratch_types=[
        pltpu.VMEM((_B_PER_W,), jnp.int32),
        pltpu.VMEM((_NCHUNK, _CHUNK), jnp.int32),
        pltpu.VMEM((_NCHUNK, _CHUNK, DIM), jnp.float32),
        pltpu.SemaphoreType.DMA,
    ],
    compiler_params=pltpu.CompilerParams(use_tc_tiling_on_sc=False),
)
def _embed_lookup(x1_hbm, emb_hbm, out_hbm, raw_v, idx_v, rows_v, sem):
    wid = lax.axis_index("s") * _NC + lax.axis_index("c")
    base = wid * _B_PER_W

    pltpu.sync_copy(x1_hbm.at[pl.ds(base, _B_PER_W)], raw_v)

    for j in range(_NCHUNK):
        for i in range(_CHUNK // _L):
            vals = raw_v[pl.ds(j * _CHUNK + i * _L, _L)]
            idx_v[j, pl.ds(i * _L, _L)] = lax.div(vals, 90)

    copies = [
        pltpu.async_copy(emb_hbm.at[idx_v.at[j]], rows_v.at[j], sem)
        for j in range(_NCHUNK)
    ]
    for c in copies:
        c.wait()

    for j in range(_NCHUNK):
        pltpu.sync_copy(
            rows_v.at[j], out_hbm.at[pl.ds(base + j * _CHUNK, _CHUNK)]
        )


def kernel(x, emb):
    x1 = x[:, 1].astype(jnp.int32)
    return _embed_lookup(x1, emb)
